# Initial kernel scaffold; baseline (speedup 1.0000x reference)
import jax
import jax.numpy as jnp
from jax.experimental import pallas as pl


def kernel(x_nchw, w1_hwio, b1, w2_hwio, b2):
    raise NotImplementedError("write your pallas kernel here")


# speedup vs baseline: 1.5029x; 1.5029x over previous
import functools

import jax
import jax.numpy as jnp
from jax.experimental import pallas as pl
from jax.experimental.pallas import tpu as pltpu


def _resblock_kernel(x_ref, w1_ref, b1_ref, w2_ref, b2_ref, out_ref, *, H, W):
    BT, C, HW = x_ref.shape

    p = jax.lax.broadcasted_iota(jnp.int32, (1, HW), 1)
    row = p // W
    col = p - row * W
    m_top = (row >= 1).astype(jnp.bfloat16)
    m_bot = (row <= H - 2).astype(jnp.bfloat16)
    m_left = (col >= 1).astype(jnp.float32)
    m_right = (col <= W - 2).astype(jnp.float32)

    w1 = w1_ref[...]
    w2 = w2_ref[...]
    b1 = b1_ref[...]
    b2 = b2_ref[...]

    def conv3x3(xb, wall, bias):
        z_up = pltpu.roll(xb, W, axis=1) * m_top
        z_dn = pltpu.roll(xb, HW - W, axis=1) * m_bot
        slab = jnp.concatenate([z_up, xb, z_dn], axis=0)
        u = jnp.dot(wall, slab, preferred_element_type=jnp.float32)
        y = (u[C:2 * C]
             + pltpu.roll(u[0:C], 1, axis=1) * m_left
             + pltpu.roll(u[2 * C:3 * C], HW - 1, axis=1) * m_right)
        return y + bias

    for b in range(BT):
        x32 = x_ref[b]
        h1 = jnp.maximum(conv3x3(x32.astype(jnp.bfloat16), w1, b1), 0.0)
        y = conv3x3(h1.astype(jnp.bfloat16), w2, b2)
        out_ref[b] = jnp.maximum(y + x32, 0.0)


def _pack_w(w_hwio):
    C = w_hwio.shape[2]
    return w_hwio.transpose(1, 3, 0, 2).reshape(3 * C, 3 * C).astype(jnp.bfloat16)


def kernel(x_nchw, w1_hwio, b1, w2_hwio, b2):
    B, C, H, W = x_nchw.shape
    HW = H * W
    bt = 1
    nb = B // bt

    xf = x_nchw.reshape(B, C, HW)
    w1m = _pack_w(w1_hwio)
    w2m = _pack_w(w2_hwio)
    b1c = b1.reshape(C, 1)
    b2c = b2.reshape(C, 1)

    kern = functools.partial(_resblock_kernel, H=H, W=W)
    out_flat = pl.pallas_call(
        kern,
        out_shape=jax.ShapeDtypeStruct((B, C, HW), x_nchw.dtype),
        grid=(nb,),
        in_specs=[
            pl.BlockSpec((bt, C, HW), lambda i: (i, 0, 0)),
            pl.BlockSpec((3 * C, 3 * C), lambda i: (0, 0)),
            pl.BlockSpec((C, 1), lambda i: (0, 0)),
            pl.BlockSpec((3 * C, 3 * C), lambda i: (0, 0)),
            pl.BlockSpec((C, 1), lambda i: (0, 0)),
        ],
        out_specs=pl.BlockSpec((bt, C, HW), lambda i: (i, 0, 0)),
        compiler_params=pltpu.CompilerParams(
            dimension_semantics=("parallel",)),
    )(xf, w1m, b1c, w2m, b2c)

    return out_flat.reshape(B, C, H, W)


# speedup vs baseline: 1.6243x; 1.0808x over previous
import functools

import jax
import jax.numpy as jnp
from jax.experimental import pallas as pl
from jax.experimental.pallas import tpu as pltpu


def _resblock_kernel(x_ref, w1_ref, b1_ref, w2_ref, b2_ref, out_ref, *, H, W):
    BT, HW, C = x_ref.shape

    p = jax.lax.broadcasted_iota(jnp.int32, (HW, 1), 0)
    pw = p % W
    m_top = (p >= W).astype(jnp.bfloat16)
    m_bot = (p < HW - W).astype(jnp.bfloat16)
    m_left = (pw >= 1).astype(jnp.float32)
    m_right = (pw <= W - 2).astype(jnp.float32)

    w1a, w1b, w1c = w1_ref[0], w1_ref[1], w1_ref[2]
    w2a, w2b, w2c = w2_ref[0], w2_ref[1], w2_ref[2]
    b1 = b1_ref[...]
    b2 = b2_ref[...]

    def conv3x3(xb, wa, wb, wc, bias):
        z_up = pltpu.roll(xb, W, axis=0) * m_top
        z_dn = pltpu.roll(xb, HW - W, axis=0) * m_bot
        u = (jnp.dot(z_up, wa, preferred_element_type=jnp.float32)
             + jnp.dot(xb, wb, preferred_element_type=jnp.float32)
             + jnp.dot(z_dn, wc, preferred_element_type=jnp.float32))
        y = (u[:, C:2 * C]
             + pltpu.roll(u[:, 0:C], 1, axis=0) * m_left
             + pltpu.roll(u[:, 2 * C:3 * C], HW - 1, axis=0) * m_right)
        return y + bias

    for b in range(BT):
        x32 = x_ref[b]
        h1 = jnp.maximum(conv3x3(x32.astype(jnp.bfloat16),
                                 w1a, w1b, w1c, b1), 0.0)
        y = conv3x3(h1.astype(jnp.bfloat16), w2a, w2b, w2c, b2)
        out_ref[b] = jnp.maximum(y + x32, 0.0)


def _pack_w(w_hwio):
    C = w_hwio.shape[2]
    return w_hwio.transpose(0, 2, 1, 3).reshape(3, C, 3 * C).astype(jnp.bfloat16)


def kernel(x_nchw, w1_hwio, b1, w2_hwio, b2):
    B, C, H, W = x_nchw.shape
    HW = H * W
    bt = 1
    nb = B // bt

    xt = x_nchw.reshape(B, C, HW).swapaxes(1, 2)
    w1m = _pack_w(w1_hwio)
    w2m = _pack_w(w2_hwio)
    b1c = b1.reshape(1, C)
    b2c = b2.reshape(1, C)

    kern = functools.partial(_resblock_kernel, H=H, W=W)
    out_t = pl.pallas_call(
        kern,
        out_shape=jax.ShapeDtypeStruct((B, HW, C), x_nchw.dtype),
        grid=(nb,),
        in_specs=[
            pl.BlockSpec((bt, HW, C), lambda i: (i, 0, 0)),
            pl.BlockSpec((3, C, 3 * C), lambda i: (0, 0, 0)),
            pl.BlockSpec((1, C), lambda i: (0, 0)),
            pl.BlockSpec((3, C, 3 * C), lambda i: (0, 0, 0)),
            pl.BlockSpec((1, C), lambda i: (0, 0)),
        ],
        out_specs=pl.BlockSpec((bt, HW, C), lambda i: (i, 0, 0)),
        compiler_params=pltpu.CompilerParams(
            dimension_semantics=("parallel",)),
    )(xt, w1m, b1c, w2m, b2c)

    return out_t.swapaxes(1, 2).reshape(B, C, H, W)


# speedup vs baseline: 2.0760x; 1.2781x over previous
import functools

import jax
import jax.numpy as jnp
from jax.experimental import pallas as pl
from jax.experimental.pallas import tpu as pltpu


def _resblock_kernel(x_ref, w1_ref, b1_ref, w2_ref, b2_ref, out_ref, *, H, W):
    BT, HW, C = x_ref.shape

    p = jax.lax.broadcasted_iota(jnp.int32, (HW, 1), 0)
    pw = p % W
    m_top = (p >= W).astype(jnp.bfloat16)
    m_bot = (p < HW - W).astype(jnp.bfloat16)
    m_left = (pw >= 1).astype(jnp.float32)
    m_right = (pw <= W - 2).astype(jnp.float32)

    w1 = w1_ref[...]
    w2 = w2_ref[...]
    b1 = b1_ref[...]
    b2 = b2_ref[...]

    def conv3x3(xb, wall, bias):
        z_up = pltpu.roll(xb, W, axis=0) * m_top
        z_dn = pltpu.roll(xb, HW - W, axis=0) * m_bot
        slab = jnp.concatenate([z_up, xb, z_dn], axis=1)
        u = jnp.dot(slab, wall, preferred_element_type=jnp.float32)
        y = (u[:, C:2 * C]
             + pltpu.roll(u[:, 0:C], 1, axis=0) * m_left
             + pltpu.roll(u[:, 2 * C:3 * C], HW - 1, axis=0) * m_right)
        return y + bias

    for b in range(BT):
        x32 = x_ref[b]
        h1 = jnp.maximum(conv3x3(x32.astype(jnp.bfloat16), w1, b1), 0.0)
        y = conv3x3(h1.astype(jnp.bfloat16), w2, b2)
        out_ref[b] = jnp.maximum(y + x32, 0.0)


def _pack_w(w_hwio):
    C = w_hwio.shape[2]
    return w_hwio.transpose(0, 2, 1, 3).reshape(3 * C, 3 * C).astype(jnp.bfloat16)


def kernel(x_nchw, w1_hwio, b1, w2_hwio, b2):
    B, C, H, W = x_nchw.shape
    HW = H * W
    bt = 1
    nb = B // bt

    xt = x_nchw.reshape(B, C, HW).swapaxes(1, 2)
    w1m = _pack_w(w1_hwio)
    w2m = _pack_w(w2_hwio)
    b1c = b1.reshape(1, C)
    b2c = b2.reshape(1, C)

    kern = functools.partial(_resblock_kernel, H=H, W=W)
    out_t = pl.pallas_call(
        kern,
        out_shape=jax.ShapeDtypeStruct((B, HW, C), x_nchw.dtype),
        grid=(nb,),
        in_specs=[
            pl.BlockSpec((bt, HW, C), lambda i: (i, 0, 0)),
            pl.BlockSpec((3 * C, 3 * C), lambda i: (0, 0)),
            pl.BlockSpec((1, C), lambda i: (0, 0)),
            pl.BlockSpec((3 * C, 3 * C), lambda i: (0, 0)),
            pl.BlockSpec((1, C), lambda i: (0, 0)),
        ],
        out_specs=pl.BlockSpec((bt, HW, C), lambda i: (i, 0, 0)),
        compiler_params=pltpu.CompilerParams(
            dimension_semantics=("parallel",)),
    )(xt, w1m, b1c, w2m, b2c)

    return out_t.swapaxes(1, 2).reshape(B, C, H, W)


# speedup vs baseline: 2.3990x; 1.1556x over previous
import functools

import jax
import jax.numpy as jnp
from jax.experimental import pallas as pl
from jax.experimental.pallas import tpu as pltpu


def _resblock_kernel(x_ref, w1_ref, b1_ref, w2_ref, b2_ref, out_ref, *, H, W):
    BT, HW, C = x_ref.shape

    p = jax.lax.broadcasted_iota(jnp.int32, (HW, 1), 0)
    pw = p % W
    m_top = (p >= W).astype(jnp.bfloat16)
    m_bot = (p < HW - W).astype(jnp.bfloat16)
    m_left = (pw >= 1).astype(jnp.float32)
    m_right = (pw <= W - 2).astype(jnp.float32)

    w1 = w1_ref[...]
    w2 = w2_ref[...]
    b1 = b1_ref[...]
    b2 = b2_ref[...]

    def conv3x3(xb, wall, bias):
        z_up = pltpu.roll(xb, W, axis=0) * m_top
        z_dn = pltpu.roll(xb, HW - W, axis=0) * m_bot
        slab = jnp.concatenate([z_up, xb, z_dn], axis=1)
        u = jnp.dot(slab, wall, preferred_element_type=jnp.float32)
        y = (u[:, C:2 * C]
             + pltpu.roll(u[:, 0:C], 1, axis=0) * m_left
             + pltpu.roll(u[:, 2 * C:3 * C], HW - 1, axis=0) * m_right)
        return y + bias

    for b in range(BT):
        x32 = x_ref[b]
        h1 = jnp.maximum(conv3x3(x32.astype(jnp.bfloat16), w1, b1), 0.0)
        y = conv3x3(h1.astype(jnp.bfloat16), w2, b2)
        out_ref[b] = jnp.maximum(y + x32, 0.0)


def _pack_w(w_hwio):
    C = w_hwio.shape[2]
    return w_hwio.transpose(0, 2, 1, 3).reshape(3 * C, 3 * C).astype(jnp.bfloat16)


def kernel(x_nchw, w1_hwio, b1, w2_hwio, b2):
    B, C, H, W = x_nchw.shape
    HW = H * W
    bt = 2
    nb = B // bt

    xt = x_nchw.reshape(B, C, HW).swapaxes(1, 2)
    w1m = _pack_w(w1_hwio)
    w2m = _pack_w(w2_hwio)
    b1c = b1.reshape(1, C)
    b2c = b2.reshape(1, C)

    kern = functools.partial(_resblock_kernel, H=H, W=W)
    out_t = pl.pallas_call(
        kern,
        out_shape=jax.ShapeDtypeStruct((B, HW, C), x_nchw.dtype),
        grid=(nb,),
        in_specs=[
            pl.BlockSpec((bt, HW, C), lambda i: (i, 0, 0)),
            pl.BlockSpec((3 * C, 3 * C), lambda i: (0, 0)),
            pl.BlockSpec((1, C), lambda i: (0, 0)),
            pl.BlockSpec((3 * C, 3 * C), lambda i: (0, 0)),
            pl.BlockSpec((1, C), lambda i: (0, 0)),
        ],
        out_specs=pl.BlockSpec((bt, HW, C), lambda i: (i, 0, 0)),
        compiler_params=pltpu.CompilerParams(
            dimension_semantics=("parallel",)),
    )(xt, w1m, b1c, w2m, b2c)

    return out_t.swapaxes(1, 2).reshape(B, C, H, W)


# speedup vs baseline: 2.5657x; 1.0695x over previous
import functools

import jax
import jax.numpy as jnp
from jax.experimental import pallas as pl
from jax.experimental.pallas import tpu as pltpu


def _resblock_kernel(x_ref, w1_ref, b1_ref, w2_ref, b2_ref, out_ref, *, H, W):
    BT, HW, C = x_ref.shape

    p = jax.lax.broadcasted_iota(jnp.int32, (HW, 1), 0)
    pw = p % W
    m_top = (p >= W).astype(jnp.bfloat16)
    m_bot = (p < HW - W).astype(jnp.bfloat16)
    m_left = (pw >= 1).astype(jnp.float32)
    m_right = (pw <= W - 2).astype(jnp.float32)

    w1 = w1_ref[...]
    w2 = w2_ref[...]
    b1 = b1_ref[...]
    b2 = b2_ref[...]

    def conv3x3(xb, wall, bias):
        z_up = pltpu.roll(xb, W, axis=0) * m_top
        z_dn = pltpu.roll(xb, HW - W, axis=0) * m_bot
        slab = jnp.concatenate([z_up, xb, z_dn], axis=1)
        u = jnp.dot(slab, wall, preferred_element_type=jnp.float32)
        y = (u[:, C:2 * C]
             + pltpu.roll(u[:, 0:C], 1, axis=0) * m_left
             + pltpu.roll(u[:, 2 * C:3 * C], HW - 1, axis=0) * m_right)
        return y + bias

    for b in range(BT):
        x32 = x_ref[b]
        h1 = jnp.maximum(conv3x3(x32.astype(jnp.bfloat16), w1, b1), 0.0)
        y = conv3x3(h1.astype(jnp.bfloat16), w2, b2)
        out_ref[b] = jnp.maximum(y + x32, 0.0)


def _pack_w(w_hwio):
    C = w_hwio.shape[2]
    return w_hwio.transpose(0, 2, 1, 3).reshape(3 * C, 3 * C).astype(jnp.bfloat16)


def kernel(x_nchw, w1_hwio, b1, w2_hwio, b2):
    B, C, H, W = x_nchw.shape
    HW = H * W
    bt = 4
    nb = B // bt

    xt = x_nchw.reshape(B, C, HW).swapaxes(1, 2)
    w1m = _pack_w(w1_hwio)
    w2m = _pack_w(w2_hwio)
    b1c = b1.reshape(1, C)
    b2c = b2.reshape(1, C)

    kern = functools.partial(_resblock_kernel, H=H, W=W)
    out_t = pl.pallas_call(
        kern,
        out_shape=jax.ShapeDtypeStruct((B, HW, C), x_nchw.dtype),
        grid=(nb,),
        in_specs=[
            pl.BlockSpec((bt, HW, C), lambda i: (i, 0, 0)),
            pl.BlockSpec((3 * C, 3 * C), lambda i: (0, 0)),
            pl.BlockSpec((1, C), lambda i: (0, 0)),
            pl.BlockSpec((3 * C, 3 * C), lambda i: (0, 0)),
            pl.BlockSpec((1, C), lambda i: (0, 0)),
        ],
        out_specs=pl.BlockSpec((bt, HW, C), lambda i: (i, 0, 0)),
        compiler_params=pltpu.CompilerParams(
            dimension_semantics=("parallel",)),
    )(xt, w1m, b1c, w2m, b2c)

    return out_t.swapaxes(1, 2).reshape(B, C, H, W)


# speedup vs baseline: 2.6114x; 1.0178x over previous
import functools

import jax
import jax.numpy as jnp
from jax.experimental import pallas as pl
from jax.experimental.pallas import tpu as pltpu


def _resblock_kernel(x_ref, w1_ref, b1_ref, w2_ref, b2_ref, out_ref, *, H, W):
    BT, HW, C = x_ref.shape

    p = jax.lax.broadcasted_iota(jnp.int32, (HW, 1), 0)
    pw = p % W
    m_top = (p >= W).astype(jnp.bfloat16)
    m_bot = (p < HW - W).astype(jnp.bfloat16)
    m_left = (pw >= 1).astype(jnp.float32)
    m_right = (pw <= W - 2).astype(jnp.float32)

    w1 = w1_ref[...]
    w2 = w2_ref[...]
    b1 = b1_ref[...]
    b2 = b2_ref[...]

    def conv3x3(xb, wall, bias):
        z_up = pltpu.roll(xb, W, axis=0) * m_top
        z_dn = pltpu.roll(xb, HW - W, axis=0) * m_bot
        slab = jnp.concatenate([z_up, xb, z_dn], axis=1)
        u = jnp.dot(slab, wall, preferred_element_type=jnp.float32)
        y = (u[:, C:2 * C]
             + pltpu.roll(u[:, 0:C], 1, axis=0) * m_left
             + pltpu.roll(u[:, 2 * C:3 * C], HW - 1, axis=0) * m_right)
        return y + bias

    for b in range(BT):
        x32 = x_ref[b]
        h1 = jnp.maximum(conv3x3(x32.astype(jnp.bfloat16), w1, b1), 0.0)
        y = conv3x3(h1.astype(jnp.bfloat16), w2, b2)
        out_ref[b] = jnp.maximum(y + x32, 0.0)


def _pack_w(w_hwio):
    C = w_hwio.shape[2]
    return w_hwio.transpose(0, 2, 1, 3).reshape(3 * C, 3 * C).astype(jnp.bfloat16)


def kernel(x_nchw, w1_hwio, b1, w2_hwio, b2):
    B, C, H, W = x_nchw.shape
    HW = H * W
    bt = 8
    nb = B // bt

    xt = x_nchw.reshape(B, C, HW).swapaxes(1, 2)
    w1m = _pack_w(w1_hwio)
    w2m = _pack_w(w2_hwio)
    b1c = b1.reshape(1, C)
    b2c = b2.reshape(1, C)

    kern = functools.partial(_resblock_kernel, H=H, W=W)
    out_t = pl.pallas_call(
        kern,
        out_shape=jax.ShapeDtypeStruct((B, HW, C), x_nchw.dtype),
        grid=(nb,),
        in_specs=[
            pl.BlockSpec((bt, HW, C), lambda i: (i, 0, 0)),
            pl.BlockSpec((3 * C, 3 * C), lambda i: (0, 0)),
            pl.BlockSpec((1, C), lambda i: (0, 0)),
            pl.BlockSpec((3 * C, 3 * C), lambda i: (0, 0)),
            pl.BlockSpec((1, C), lambda i: (0, 0)),
        ],
        out_specs=pl.BlockSpec((bt, HW, C), lambda i: (i, 0, 0)),
        compiler_params=pltpu.CompilerParams(
            dimension_semantics=("parallel",)),
    )(xt, w1m, b1c, w2m, b2c)

    return out_t.swapaxes(1, 2).reshape(B, C, H, W)


# speedup vs baseline: 2.8779x; 1.1021x over previous
import functools

import jax
import jax.numpy as jnp
from jax.experimental import pallas as pl
from jax.experimental.pallas import tpu as pltpu


def _resblock_kernel(x_ref, w1_ref, b1_ref, w2_ref, b2_ref, out_ref, *, H, W):
    BT, HW, C = x_ref.shape

    p = jax.lax.broadcasted_iota(jnp.int32, (HW, 1), 0)
    pw = p % W
    m_top = (p >= W).astype(jnp.bfloat16)
    m_bot = (p < HW - W).astype(jnp.bfloat16)
    m_left = (pw >= 1).astype(jnp.float32)
    m_right = (pw <= W - 2).astype(jnp.float32)

    w1 = w1_ref[...]
    w2 = w2_ref[...]
    b1 = b1_ref[...]
    b2 = b2_ref[...]

    def conv3x3(xb, wall, bias):
        z_up = pltpu.roll(xb, W, axis=0) * m_top
        z_dn = pltpu.roll(xb, HW - W, axis=0) * m_bot
        slab = jnp.concatenate([z_up, xb, z_dn], axis=1)
        u = jnp.dot(slab, wall, preferred_element_type=jnp.float32)
        y = (u[:, C:2 * C]
             + pltpu.roll(u[:, 0:C], 1, axis=0) * m_left
             + pltpu.roll(u[:, 2 * C:3 * C], HW - 1, axis=0) * m_right)
        return y + bias

    h1 = []
    for b in range(BT):
        h1.append(jnp.maximum(
            conv3x3(x_ref[b].astype(jnp.bfloat16), w1, b1),
            0.0).astype(jnp.bfloat16))
    for b in range(BT):
        y = conv3x3(h1[b], w2, b2)
        out_ref[b] = jnp.maximum(y + x_ref[b], 0.0)


def _pack_w(w_hwio):
    C = w_hwio.shape[2]
    return w_hwio.transpose(0, 2, 1, 3).reshape(3 * C, 3 * C).astype(jnp.bfloat16)


def kernel(x_nchw, w1_hwio, b1, w2_hwio, b2):
    B, C, H, W = x_nchw.shape
    HW = H * W
    bt = 8 if B % 8 == 0 else 1
    nb = B // bt

    xt = x_nchw.reshape(B, C, HW).swapaxes(1, 2)
    w1m = _pack_w(w1_hwio)
    w2m = _pack_w(w2_hwio)
    b1c = b1.reshape(1, C)
    b2c = b2.reshape(1, C)

    kern = functools.partial(_resblock_kernel, H=H, W=W)
    out_t = pl.pallas_call(
        kern,
        out_shape=jax.ShapeDtypeStruct((B, HW, C), x_nchw.dtype),
        grid=(nb,),
        in_specs=[
            pl.BlockSpec((bt, HW, C), lambda i: (i, 0, 0)),
            pl.BlockSpec((3 * C, 3 * C), lambda i: (0, 0)),
            pl.BlockSpec((1, C), lambda i: (0, 0)),
            pl.BlockSpec((3 * C, 3 * C), lambda i: (0, 0)),
            pl.BlockSpec((1, C), lambda i: (0, 0)),
        ],
        out_specs=pl.BlockSpec((bt, HW, C), lambda i: (i, 0, 0)),
        compiler_params=pltpu.CompilerParams(
            dimension_semantics=("parallel",)),
    )(xt, w1m, b1c, w2m, b2c)

    return out_t.swapaxes(1, 2).reshape(B, C, H, W)
